# Initial kernel scaffold; baseline (speedup 1.0000x reference)
import functools

import jax
import jax.numpy as jnp
from jax import lax
from jax.experimental import pallas as pl
from jax.experimental.pallas import tpu as pltpu
from jax.experimental.pallas import tpu_sc as plsc

_NC = 2
_NS = 16
_LANES = 16

_SQRT_HALF = 0.7071067811865476


def _gelu(h):
    return 0.5 * h * (1.0 + lax.erf(h * _SQRT_HALF))


def _dot_t(a, w):
    return lax.dot_general(a, w, (((1,), (1,)), ((), ())),
                           preferred_element_type=jnp.float32)


def _pre_body(x_ref, wd_ref, bd_ref, ws_ref, bs_ref, h0_ref, sc_ref):
    x = x_ref[...]
    h0_ref[...] = _gelu(_dot_t(x, wd_ref[...]) + bd_ref[...])
    sc_ref[...] = _dot_t(x, ws_ref[...]) + bs_ref[...]


def _ln(t, g, b):
    mu = jnp.mean(t, axis=-1, keepdims=True)
    var = jnp.mean((t - mu) ** 2, axis=-1, keepdims=True)
    return (t - mu) * lax.rsqrt(var + 1e-5) * g + b


def _sage_ln(p_ref, c_ref, h_ref, wl_ref, bl_ref, wr_ref, g_ref, b_ref):
    p = p_ref[0] + p_ref[1]
    cnt = c_ref[0, :, 0:1] + c_ref[1, :, 0:1]
    agg = p / jnp.maximum(cnt, 1.0)
    t = _dot_t(agg, wl_ref[...]) + bl_ref[...] + _dot_t(h_ref[...], wr_ref[...])
    return _ln(t, g_ref[...], b_ref[...])


def _mid_body(p_ref, c_ref, h_ref, wl_ref, bl_ref, wr_ref, g_ref, b_ref, o_ref):
    o_ref[...] = _gelu(_sage_ln(p_ref, c_ref, h_ref, wl_ref, bl_ref, wr_ref,
                                g_ref, b_ref))


def _post_body(p_ref, c_ref, h_ref, wl_ref, bl_ref, wr_ref, g_ref, b_ref,
               sc_ref, o_ref):
    t = _sage_ln(p_ref, c_ref, h_ref, wl_ref, bl_ref, wr_ref, g_ref, b_ref)
    o_ref[...] = _gelu(t + sc_ref[...])


def _row_spec(r, d):
    return pl.BlockSpec((r, d), lambda i: (i, 0))


def _full_spec(shape):
    nd = len(shape)
    return pl.BlockSpec(shape, lambda i: (0,) * nd)


@functools.partial(jax.jit, static_argnames=("n", "d", "r"))
def _tc_pre(x, wd, bd, ws, bs, *, n, d, r):
    grid = (n // r,)
    out = [jax.ShapeDtypeStruct((n, d), jnp.float32)] * 2
    return pl.pallas_call(
        _pre_body,
        grid=grid,
        in_specs=[_row_spec(r, d), _full_spec((d, d)), _full_spec((1, d)),
                  _full_spec((d, d)), _full_spec((1, d))],
        out_specs=[_row_spec(r, d), _row_spec(r, d)],
        out_shape=out,
    )(x, wd, bd.reshape(1, d), ws, bs.reshape(1, d))


def _sage_specs(r, d):
    return [
        pl.BlockSpec((_NC, r, d), lambda i: (0, i, 0)),
        pl.BlockSpec((_NC, r, _LANES), lambda i: (0, i, 0)),
        _row_spec(r, d),
        _full_spec((d, d)), _full_spec((1, d)), _full_spec((d, d)),
        _full_spec((1, d)), _full_spec((1, d)),
    ]


@functools.partial(jax.jit, static_argnames=("n", "d", "r"))
def _tc_mid(p, c, h, wl, bl, wr, g, b, *, n, d, r):
    return pl.pallas_call(
        _mid_body,
        grid=(n // r,),
        in_specs=_sage_specs(r, d),
        out_specs=_row_spec(r, d),
        out_shape=jax.ShapeDtypeStruct((n, d), jnp.float32),
    )(p, c, h, wl, bl.reshape(1, d), wr, g.reshape(1, d), b.reshape(1, d))


@functools.partial(jax.jit, static_argnames=("n", "d", "r"))
def _tc_post(p, c, h, wl, bl, wr, g, b, sc, *, n, d, r):
    return pl.pallas_call(
        _post_body,
        grid=(n // r,),
        in_specs=_sage_specs(r, d) + [_row_spec(r, d)],
        out_specs=_row_spec(r, d),
        out_shape=jax.ShapeDtypeStruct((n, d), jnp.float32),
    )(p, c, h, wl, bl.reshape(1, d), wr, g.reshape(1, d), b.reshape(1, d), sc)


_CHUNK = 128


@functools.partial(functools.lru_cache, maxsize=None)
def _make_sc_agg(n, e, d, with_counts):
    nw = _NC * _NS
    assert e % nw == 0 and n % _NS == 0 and d % _LANES == 0
    epw = e // nw
    nch = epw // _CHUNK
    tail = epw - nch * _CHUNK
    rps = n // _NS
    zrows = 128 if rps % 128 == 0 else (64 if rps % 64 == 0 else 25)
    assert rps % zrows == 0 and tail % 8 == 0

    mesh = plsc.VectorSubcoreMesh(core_axis_name="c", subcore_axis_name="s")

    out_type = [jax.ShapeDtypeStruct((_NC, n, d), jnp.float32)]
    scratch = [
        pltpu.VMEM_SHARED((n, d), jnp.float32),
        pltpu.VMEM((epw,), jnp.int32),
        pltpu.VMEM((epw,), jnp.int32),
        pltpu.VMEM((_CHUNK,), jnp.int32),
        pltpu.VMEM((_CHUNK, d), jnp.float32),
        pltpu.VMEM((zrows, d), jnp.float32),
        pltpu.SemaphoreType.DMA,
    ]
    if with_counts:
        out_type.append(jax.ShapeDtypeStruct((_NC, n, _LANES), jnp.float32))
        scratch.append(pltpu.VMEM_SHARED((n, _LANES), jnp.float32))
        scratch.append(pltpu.VMEM((_CHUNK, _LANES), jnp.float32))
    if tail:
        scratch.append(pltpu.VMEM((tail,), jnp.int32))
        scratch.append(pltpu.VMEM((tail, d), jnp.float32))

    def body(h_hbm, src_hbm, dst_hbm, *outs_scratch):
        it = iter(outs_scratch)
        psum_hbm = next(it)
        pcnt_hbm = next(it) if with_counts else None
        acc = next(it)
        sidx = next(it)
        didx = next(it)
        dchunk = next(it)
        rows = next(it)
        zbuf = next(it)
        sem = next(it)
        if with_counts:
            cntacc = next(it)
            ones = next(it)
        if tail:
            dtail = next(it)
            rtail = next(it)

        ci = lax.axis_index("c")
        si = lax.axis_index("s")
        w = ci * _NS + si
        ebase = w * epw
        rbase = si * rps

        def fill_z(i, _):
            for j in range(d // _LANES):
                zbuf[i, pl.ds(j * _LANES, _LANES)] = jnp.zeros(
                    (_LANES,), jnp.float32)
            return 0
        lax.fori_loop(0, zrows, fill_z, 0)
        if with_counts:
            def fill_o(i, _):
                ones[i, :] = jnp.ones((_LANES,), jnp.float32)
                return 0
            lax.fori_loop(0, _CHUNK, fill_o, 0)

        for k in range(rps // zrows):
            pltpu.sync_copy(zbuf, acc.at[pl.ds(rbase + k * zrows, zrows)])
            if with_counts:
                pltpu.sync_copy(zbuf.at[:, pl.ds(0, _LANES)],
                                cntacc.at[pl.ds(rbase + k * zrows, zrows)])

        pltpu.sync_copy(src_hbm.at[pl.ds(ebase, epw)], sidx)
        pltpu.sync_copy(dst_hbm.at[pl.ds(ebase, epw)], didx)

        plsc.subcore_barrier()

        def chunk(i, _):
            off = i * _CHUNK
            pltpu.sync_copy(didx.at[pl.ds(off, _CHUNK)], dchunk)
            pltpu.async_copy(h_hbm.at[sidx.at[pl.ds(off, _CHUNK)]], rows,
                             sem).wait()
            pltpu.sync_copy(rows, acc.at[dchunk], add=True)
            if with_counts:
                pltpu.sync_copy(ones, cntacc.at[dchunk], add=True)
            return 0
        lax.fori_loop(0, nch, chunk, 0)

        if tail:
            off = nch * _CHUNK
            pltpu.sync_copy(didx.at[pl.ds(off, tail)], dtail)
            pltpu.async_copy(h_hbm.at[sidx.at[pl.ds(off, tail)]], rtail,
                             sem).wait()
            pltpu.sync_copy(rtail, acc.at[dtail], add=True)
            if with_counts:
                pltpu.sync_copy(ones.at[pl.ds(0, tail)], cntacc.at[dtail],
                                add=True)

        plsc.subcore_barrier()

        pltpu.sync_copy(acc.at[pl.ds(rbase, rps)],
                        psum_hbm.at[ci, pl.ds(rbase, rps)])
        if with_counts:
            pltpu.sync_copy(cntacc.at[pl.ds(rbase, rps)],
                            pcnt_hbm.at[ci, pl.ds(rbase, rps)])

    return pl.kernel(body, out_type=out_type, mesh=mesh,
                     scratch_types=scratch)


def kernel(x, edges, W_down, b_down, W_sc, b_sc, Wl1, bl1, Wr1, ln1_g, ln1_b,
           Wl2, bl2, Wr2, ln2_g, ln2_b):
    n, d = x.shape
    e = edges.shape[1]
    src = edges[0]
    dst = edges[1]
    r = 2000 if n % 2000 == 0 else n

    h0, shortcut = _tc_pre(x, W_down, b_down, W_sc, b_sc, n=n, d=d, r=r)

    agg1 = _make_sc_agg(n, e, d, True)
    p1, c1 = agg1(h0, src, dst)
    h1 = _tc_mid(p1, c1, h0, Wl1, bl1, Wr1, ln1_g, ln1_b, n=n, d=d, r=r)

    agg2 = _make_sc_agg(n, e, d, False)
    (p2,) = agg2(h1, src, dst)
    return _tc_post(p2, c1, h1, Wl2, bl2, Wr2, ln2_g, ln2_b, shortcut,
                    n=n, d=d, r=r)


# speedup vs baseline: 5.9801x; 5.9801x over previous
import functools

import jax
import jax.numpy as jnp
from jax import lax
from jax.experimental import pallas as pl
from jax.experimental.pallas import tpu as pltpu
from jax.experimental.pallas import tpu_sc as plsc

_NC = 2
_NS = 16
_LANES = 16

_SQRT_HALF = 0.7071067811865476


def _gelu(h):
    return 0.5 * h * (1.0 + lax.erf(h * _SQRT_HALF))


def _dot_t(a, w):
    return lax.dot_general(a, w, (((1,), (1,)), ((), ())),
                           preferred_element_type=jnp.float32)


def _pre_body(x_ref, wd_ref, bd_ref, ws_ref, bs_ref, h0_ref, sc_ref):
    x = x_ref[...]
    h0_ref[...] = _gelu(_dot_t(x, wd_ref[...]) + bd_ref[...])
    sc_ref[...] = _dot_t(x, ws_ref[...]) + bs_ref[...]


def _ln(t, g, b):
    mu = jnp.mean(t, axis=-1, keepdims=True)
    var = jnp.mean((t - mu) ** 2, axis=-1, keepdims=True)
    return (t - mu) * lax.rsqrt(var + 1e-5) * g + b


def _sage_ln(p_ref, c_ref, h_ref, wl_ref, bl_ref, wr_ref, g_ref, b_ref):
    p = p_ref[0] + p_ref[1]
    cnt = c_ref[0, :, 0:1] + c_ref[1, :, 0:1]
    agg = p / jnp.maximum(cnt, 1.0)
    t = _dot_t(agg, wl_ref[...]) + bl_ref[...] + _dot_t(h_ref[...], wr_ref[...])
    return _ln(t, g_ref[...], b_ref[...])


def _mid_body(p_ref, c_ref, h_ref, wl_ref, bl_ref, wr_ref, g_ref, b_ref, o_ref):
    o_ref[...] = _gelu(_sage_ln(p_ref, c_ref, h_ref, wl_ref, bl_ref, wr_ref,
                                g_ref, b_ref))


def _post_body(p_ref, c_ref, h_ref, wl_ref, bl_ref, wr_ref, g_ref, b_ref,
               sc_ref, o_ref):
    t = _sage_ln(p_ref, c_ref, h_ref, wl_ref, bl_ref, wr_ref, g_ref, b_ref)
    o_ref[...] = _gelu(t + sc_ref[...])


def _row_spec(r, d):
    return pl.BlockSpec((r, d), lambda i: (i, 0))


def _full_spec(shape):
    nd = len(shape)
    return pl.BlockSpec(shape, lambda i: (0,) * nd)


@functools.partial(jax.jit, static_argnames=("n", "d", "r"))
def _tc_pre(x, wd, bd, ws, bs, *, n, d, r):
    grid = (n // r,)
    out = [jax.ShapeDtypeStruct((n, d), jnp.float32)] * 2
    return pl.pallas_call(
        _pre_body,
        grid=grid,
        in_specs=[_row_spec(r, d), _full_spec((d, d)), _full_spec((1, d)),
                  _full_spec((d, d)), _full_spec((1, d))],
        out_specs=[_row_spec(r, d), _row_spec(r, d)],
        out_shape=out,
    )(x, wd, bd.reshape(1, d), ws, bs.reshape(1, d))


def _sage_specs(r, d):
    return [
        pl.BlockSpec((_NC, r, d), lambda i: (0, i, 0)),
        pl.BlockSpec((_NC, r, _LANES), lambda i: (0, i, 0)),
        _row_spec(r, d),
        _full_spec((d, d)), _full_spec((1, d)), _full_spec((d, d)),
        _full_spec((1, d)), _full_spec((1, d)),
    ]


@functools.partial(jax.jit, static_argnames=("n", "d", "r"))
def _tc_mid(p, c, h, wl, bl, wr, g, b, *, n, d, r):
    return pl.pallas_call(
        _mid_body,
        grid=(n // r,),
        in_specs=_sage_specs(r, d),
        out_specs=_row_spec(r, d),
        out_shape=jax.ShapeDtypeStruct((n, d), jnp.float32),
    )(p, c, h, wl, bl.reshape(1, d), wr, g.reshape(1, d), b.reshape(1, d))


@functools.partial(jax.jit, static_argnames=("n", "d", "r"))
def _tc_post(p, c, h, wl, bl, wr, g, b, sc, *, n, d, r):
    return pl.pallas_call(
        _post_body,
        grid=(n // r,),
        in_specs=_sage_specs(r, d) + [_row_spec(r, d)],
        out_specs=_row_spec(r, d),
        out_shape=jax.ShapeDtypeStruct((n, d), jnp.float32),
    )(p, c, h, wl, bl.reshape(1, d), wr, g.reshape(1, d), b.reshape(1, d), sc)


_CHUNK = 128


def _sc_partition(n, e):
    nw = _NC * _NS
    assert e % nw == 0
    epw = e // nw
    nch = epw // _CHUNK
    tail = epw - nch * _CHUNK
    npad = -(-n // (_NS * 128)) * (_NS * 128)
    rps = npad // _NS
    zrows = 128
    assert tail % 8 == 0 and rps % zrows == 0
    return epw, nch, tail, npad, rps, zrows


def _mesh():
    return plsc.VectorSubcoreMesh(core_axis_name="c", subcore_axis_name="s",
                                  num_cores=_NC, num_subcores=_NS)


@functools.lru_cache(maxsize=None)
def _make_sc_agg(n, e, d):
    assert d % _LANES == 0
    epw, nch, tail, npad, rps, zrows = _sc_partition(n, e)

    scratch = [
        pltpu.VMEM_SHARED((npad, d), jnp.float32),
        pltpu.VMEM((_CHUNK,), jnp.int32),
        pltpu.VMEM((_CHUNK,), jnp.int32),
        pltpu.VMEM((_CHUNK, d), jnp.float32),
        pltpu.VMEM((zrows, d), jnp.float32),
        pltpu.SemaphoreType.DMA,
    ]
    if tail:
        scratch.append(pltpu.VMEM((tail,), jnp.int32))
        scratch.append(pltpu.VMEM((tail,), jnp.int32))
        scratch.append(pltpu.VMEM((tail, d), jnp.float32))

    def body(h_hbm, src_hbm, dst_hbm, psum_hbm, *scr):
        it = iter(scr)
        acc = next(it)
        schunk = next(it)
        dchunk = next(it)
        rows = next(it)
        zbuf = next(it)
        sem = next(it)
        if tail:
            stail = next(it)
            dtail = next(it)
            rtail = next(it)

        ci = lax.axis_index("c")
        si = lax.axis_index("s")
        ebase = (ci * _NS + si) * epw
        rbase = pl.multiple_of(si * rps, 128)

        def fill_z(i, _):
            for j in range(d // _LANES):
                zbuf[i, pl.ds(j * _LANES, _LANES)] = jnp.zeros(
                    (_LANES,), jnp.float32)
            return 0
        lax.fori_loop(0, zrows, fill_z, 0)

        for k in range(rps // zrows):
            pltpu.sync_copy(zbuf, acc.at[pl.ds(rbase + k * zrows, zrows)])

        plsc.subcore_barrier()

        def chunk(i, _):
            off = ebase + i * _CHUNK
            pltpu.sync_copy(src_hbm.at[pl.ds(off, _CHUNK)], schunk)
            pltpu.sync_copy(dst_hbm.at[pl.ds(off, _CHUNK)], dchunk)
            pltpu.async_copy(h_hbm.at[schunk], rows, sem).wait()
            pltpu.sync_copy(rows, acc.at[dchunk], add=True)
            return 0
        lax.fori_loop(0, nch, chunk, 0)

        if tail:
            off = ebase + nch * _CHUNK
            pltpu.sync_copy(src_hbm.at[pl.ds(off, tail)], stail)
            pltpu.sync_copy(dst_hbm.at[pl.ds(off, tail)], dtail)
            pltpu.async_copy(h_hbm.at[stail], rtail, sem).wait()
            pltpu.sync_copy(rtail, acc.at[dtail], add=True)

        plsc.subcore_barrier()

        pltpu.sync_copy(acc.at[pl.ds(rbase, rps)],
                        psum_hbm.at[ci, pl.ds(rbase, rps)])

    return pl.kernel(
        body, out_type=jax.ShapeDtypeStruct((_NC, npad, d), jnp.float32),
        mesh=_mesh(), scratch_types=scratch)


@functools.lru_cache(maxsize=None)
def _make_sc_cnt(n, e):
    epw, nch, tail, npad, rps, zrows = _sc_partition(n, e)

    scratch = [
        pltpu.VMEM_SHARED((npad,), jnp.float32),
        pltpu.VMEM((_CHUNK,), jnp.int32),
        pltpu.VMEM((_CHUNK,), jnp.float32),
        pltpu.VMEM((rps,), jnp.float32),
    ]
    if tail:
        scratch.append(pltpu.VMEM((tail,), jnp.int32))

    def body(dst_hbm, pcnt_hbm, *scr):
        it = iter(scr)
        cntacc = next(it)
        dchunk = next(it)
        ones = next(it)
        zcnt = next(it)
        if tail:
            dtail = next(it)

        ci = lax.axis_index("c")
        si = lax.axis_index("s")
        ebase = (ci * _NS + si) * epw
        rbase = pl.multiple_of(si * rps, 128)

        def fill_o(i, _):
            ones[pl.ds(i * _LANES, _LANES)] = jnp.ones((_LANES,), jnp.float32)
            return 0
        lax.fori_loop(0, _CHUNK // _LANES, fill_o, 0)

        def fill_zc(i, _):
            zcnt[pl.ds(i * _LANES, _LANES)] = jnp.zeros((_LANES,), jnp.float32)
            return 0
        lax.fori_loop(0, rps // _LANES, fill_zc, 0)

        pltpu.sync_copy(zcnt, cntacc.at[pl.ds(rbase, rps)])

        plsc.subcore_barrier()

        def chunk(i, _):
            off = ebase + i * _CHUNK
            pltpu.sync_copy(dst_hbm.at[pl.ds(off, _CHUNK)], dchunk)
            pltpu.sync_copy(ones, cntacc.at[dchunk], add=True)
            return 0
        lax.fori_loop(0, nch, chunk, 0)

        if tail:
            off = ebase + nch * _CHUNK
            pltpu.sync_copy(dst_hbm.at[pl.ds(off, tail)], dtail)
            pltpu.sync_copy(ones.at[pl.ds(0, tail)], cntacc.at[dtail],
                            add=True)

        plsc.subcore_barrier()

        pltpu.sync_copy(cntacc.at[pl.ds(rbase, rps)],
                        pcnt_hbm.at[ci, pl.ds(rbase, rps)])

    return pl.kernel(
        body, out_type=jax.ShapeDtypeStruct((_NC, npad), jnp.float32),
        mesh=_mesh(), scratch_types=scratch)


def kernel(x, edges, W_down, b_down, W_sc, b_sc, Wl1, bl1, Wr1, ln1_g, ln1_b,
           Wl2, bl2, Wr2, ln2_g, ln2_b):
    n, d = x.shape
    e = edges.shape[1]
    src = edges[0]
    dst = edges[1]
    r = 2000 if n % 2000 == 0 else n

    h0, shortcut = _tc_pre(x, W_down, b_down, W_sc, b_sc, n=n, d=d, r=r)

    cnt = _make_sc_cnt(n, e)(dst)
    c1 = jnp.broadcast_to(cnt[:, :, None], cnt.shape + (_LANES,))
    p1 = _make_sc_agg(n, e, d)(h0, src, dst)
    h1 = _tc_mid(p1, c1, h0, Wl1, bl1, Wr1, ln1_g, ln1_b, n=n, d=d, r=r)

    p2 = _make_sc_agg(n, e, d)(h1, src, dst)
    return _tc_post(p2, c1, h1, Wl2, bl2, Wr2, ln2_g, ln2_b, shortcut,
                    n=n, d=d, r=r)


# speedup vs baseline: 8.7402x; 1.4616x over previous
import functools

import jax
import jax.numpy as jnp
from jax import lax
from jax.experimental import pallas as pl
from jax.experimental.pallas import tpu as pltpu
from jax.experimental.pallas import tpu_sc as plsc

_NC = 2
_NS = 16
_LANES = 16

_SQRT_HALF = 0.7071067811865476


def _gelu(h):
    return 0.5 * h * (1.0 + lax.erf(h * _SQRT_HALF))


def _dot_t(a, w):
    return lax.dot_general(a, w, (((1,), (1,)), ((), ())),
                           preferred_element_type=jnp.float32)


def _pre_body(x_ref, wd_ref, bd_ref, ws_ref, bs_ref, h0_ref, sc_ref):
    x = x_ref[...]
    h0_ref[...] = _gelu(_dot_t(x, wd_ref[...]) + bd_ref[...])
    sc_ref[...] = _dot_t(x, ws_ref[...]) + bs_ref[...]


def _ln(t, g, b):
    mu = jnp.mean(t, axis=-1, keepdims=True)
    var = jnp.mean((t - mu) ** 2, axis=-1, keepdims=True)
    return (t - mu) * lax.rsqrt(var + 1e-5) * g + b


def _sage_ln(p_ref, c_ref, h_ref, wl_ref, bl_ref, wr_ref, g_ref, b_ref):
    p = p_ref[0] + p_ref[1]
    cnt = c_ref[0, :, 0:1] + c_ref[1, :, 0:1]
    agg = p / jnp.maximum(cnt, 1.0)
    t = _dot_t(agg, wl_ref[...]) + bl_ref[...] + _dot_t(h_ref[...], wr_ref[...])
    return _ln(t, g_ref[...], b_ref[...])


def _mid_body(p_ref, c_ref, h_ref, wl_ref, bl_ref, wr_ref, g_ref, b_ref, o_ref):
    o_ref[...] = _gelu(_sage_ln(p_ref, c_ref, h_ref, wl_ref, bl_ref, wr_ref,
                                g_ref, b_ref))


def _post_body(p_ref, c_ref, h_ref, wl_ref, bl_ref, wr_ref, g_ref, b_ref,
               sc_ref, o_ref):
    t = _sage_ln(p_ref, c_ref, h_ref, wl_ref, bl_ref, wr_ref, g_ref, b_ref)
    o_ref[...] = _gelu(t + sc_ref[...])


def _row_spec(r, d):
    return pl.BlockSpec((r, d), lambda i: (i, 0))


def _full_spec(shape):
    nd = len(shape)
    return pl.BlockSpec(shape, lambda i: (0,) * nd)


@functools.partial(jax.jit, static_argnames=("n", "d", "r"))
def _tc_pre(x, wd, bd, ws, bs, *, n, d, r):
    grid = (n // r,)
    out = [jax.ShapeDtypeStruct((n, d), jnp.float32)] * 2
    return pl.pallas_call(
        _pre_body,
        grid=grid,
        in_specs=[_row_spec(r, d), _full_spec((d, d)), _full_spec((1, d)),
                  _full_spec((d, d)), _full_spec((1, d))],
        out_specs=[_row_spec(r, d), _row_spec(r, d)],
        out_shape=out,
    )(x, wd, bd.reshape(1, d), ws, bs.reshape(1, d))


def _sage_specs(r, d):
    return [
        pl.BlockSpec((_NC, r, d), lambda i: (0, i, 0)),
        pl.BlockSpec((_NC, r, _LANES), lambda i: (0, i, 0)),
        _row_spec(r, d),
        _full_spec((d, d)), _full_spec((1, d)), _full_spec((d, d)),
        _full_spec((1, d)), _full_spec((1, d)),
    ]


@functools.partial(jax.jit, static_argnames=("n", "d", "r"))
def _tc_mid(p, c, h, wl, bl, wr, g, b, *, n, d, r):
    return pl.pallas_call(
        _mid_body,
        grid=(n // r,),
        in_specs=_sage_specs(r, d),
        out_specs=_row_spec(r, d),
        out_shape=jax.ShapeDtypeStruct((n, d), jnp.float32),
    )(p, c, h, wl, bl.reshape(1, d), wr, g.reshape(1, d), b.reshape(1, d))


@functools.partial(jax.jit, static_argnames=("n", "d", "r"))
def _tc_post(p, c, h, wl, bl, wr, g, b, sc, *, n, d, r):
    return pl.pallas_call(
        _post_body,
        grid=(n // r,),
        in_specs=_sage_specs(r, d) + [_row_spec(r, d)],
        out_specs=_row_spec(r, d),
        out_shape=jax.ShapeDtypeStruct((n, d), jnp.float32),
    )(p, c, h, wl, bl.reshape(1, d), wr, g.reshape(1, d), b.reshape(1, d), sc)


_CHUNK = 128


def _sc_partition(n, e):
    nw = _NC * _NS
    assert e % _CHUNK == 0
    ncht = e // _CHUNK
    ncb = ncht // nw
    nex = ncht - ncb * nw
    npad = -(-n // (_NS * 128)) * (_NS * 128)
    rps = npad // _NS
    assert ncb % 2 == 0 and ncb >= 4
    return ncb, nex, npad, rps


def _mesh():
    return plsc.VectorSubcoreMesh(core_axis_name="c", subcore_axis_name="s",
                                  num_cores=_NC, num_subcores=_NS)


@functools.lru_cache(maxsize=None)
def _make_sc_agg(n, e, d):
    assert d % _LANES == 0
    ncb, nex, npad, rps = _sc_partition(n, e)
    zrows = 64
    nw = _NC * _NS

    out_type = jax.ShapeDtypeStruct((_NC, npad, d), jnp.float32)
    scratch = [
        pltpu.VMEM_SHARED((npad, d), jnp.float32),
        pltpu.VMEM((_CHUNK,), jnp.int32),
        pltpu.VMEM((_CHUNK,), jnp.int32),
        pltpu.VMEM((_CHUNK,), jnp.int32),
        pltpu.VMEM((_CHUNK,), jnp.int32),
        pltpu.VMEM((_CHUNK, d), jnp.float32),
        pltpu.VMEM((_CHUNK, d), jnp.float32),
        pltpu.VMEM((zrows, d), jnp.float32),
        pltpu.SemaphoreType.DMA,
        pltpu.SemaphoreType.DMA,
    ]

    def body(h_hbm, src_hbm, dst_hbm, *rest):
        it = iter(rest)
        psum_hbm = next(it)
        acc = next(it)
        s0 = next(it)
        d0 = next(it)
        s1 = next(it)
        d1 = next(it)
        rows0 = next(it)
        rows1 = next(it)
        zbuf = next(it)
        gsem0 = next(it)
        gsem1 = next(it)

        ci = lax.axis_index("c")
        si = lax.axis_index("s")
        w = ci * _NS + si
        cbase = w * ncb
        rbase = pl.multiple_of(si * rps, 128)

        def fill_z(i, _):
            for j in range(d // _LANES):
                zbuf[i, pl.ds(j * _LANES, _LANES)] = jnp.zeros(
                    (_LANES,), jnp.float32)
            return 0
        lax.fori_loop(0, zrows, fill_z, 0)

        for k in range(rps // zrows):
            pltpu.sync_copy(zbuf, acc.at[pl.ds(rbase + k * zrows, zrows)])

        plsc.subcore_barrier()

        def load_idx(sref, dref, c):
            off = c * _CHUNK
            pltpu.sync_copy(src_hbm.at[pl.ds(off, _CHUNK)], sref)
            pltpu.sync_copy(dst_hbm.at[pl.ds(off, _CHUNK)], dref)

        def gather(sref, rref, sem):
            pltpu.async_copy(h_hbm.at[sref], rref, sem)

        def wait(sref, rref, sem):
            pltpu.make_async_copy(h_hbm.at[sref], rref, sem).wait()

        def scatter(rref, dref):
            pltpu.sync_copy(rref, acc.at[dref], add=True)

        load_idx(s0, d0, cbase)
        gather(s0, rows0, gsem0)

        def pair(j, _):
            a = cbase + 2 * j
            load_idx(s1, d1, a + 1)
            gather(s1, rows1, gsem1)
            wait(s0, rows0, gsem0)
            scatter(rows0, d0)
            load_idx(s0, d0, a + 2)
            wait(s1, rows1, gsem1)
            gather(s0, rows0, gsem0)
            scatter(rows1, d1)
            return 0
        lax.fori_loop(0, ncb // 2 - 1, pair, 0)

        load_idx(s1, d1, cbase + ncb - 1)
        gather(s1, rows1, gsem1)
        wait(s0, rows0, gsem0)
        scatter(rows0, d0)
        wait(s1, rows1, gsem1)
        scatter(rows1, d1)

        if nex:
            @pl.when(w < nex)
            def _extra():
                load_idx(s0, d0, ncb * nw + w)
                gather(s0, rows0, gsem0)
                wait(s0, rows0, gsem0)
                scatter(rows0, d0)

        plsc.subcore_barrier()

        pltpu.sync_copy(acc.at[pl.ds(rbase, rps)],
                        psum_hbm.at[ci, pl.ds(rbase, rps)])

    return pl.kernel(body, out_type=out_type, mesh=_mesh(),
                     scratch_types=scratch)


@functools.lru_cache(maxsize=None)
def _make_sc_cnt(n, e):
    ncb, nex, npad, rps = _sc_partition(n, e)
    nw = _NC * _NS
    epw = (e // _CHUNK // nw) * _CHUNK
    nch = epw // _CHUNK
    tail = 0

    scratch = [
        pltpu.VMEM_SHARED((npad,), jnp.float32),
        pltpu.VMEM((_CHUNK,), jnp.int32),
        pltpu.VMEM((_CHUNK,), jnp.float32),
        pltpu.VMEM((rps,), jnp.float32),
    ]

    def body(dst_hbm, pcnt_hbm, *scr):
        it = iter(scr)
        cntacc = next(it)
        dchunk = next(it)
        ones = next(it)
        zcnt = next(it)

        ci = lax.axis_index("c")
        si = lax.axis_index("s")
        w = ci * _NS + si
        ebase = w * epw
        rbase = pl.multiple_of(si * rps, 128)

        def fill_o(i, _):
            ones[pl.ds(i * _LANES, _LANES)] = jnp.ones((_LANES,), jnp.float32)
            return 0
        lax.fori_loop(0, _CHUNK // _LANES, fill_o, 0)

        def fill_zc(i, _):
            zcnt[pl.ds(i * _LANES, _LANES)] = jnp.zeros((_LANES,), jnp.float32)
            return 0
        lax.fori_loop(0, rps // _LANES, fill_zc, 0)

        pltpu.sync_copy(zcnt, cntacc.at[pl.ds(rbase, rps)])

        plsc.subcore_barrier()

        def chunk(i, _):
            off = ebase + i * _CHUNK
            pltpu.sync_copy(dst_hbm.at[pl.ds(off, _CHUNK)], dchunk)
            pltpu.sync_copy(ones, cntacc.at[dchunk], add=True)
            return 0
        lax.fori_loop(0, nch, chunk, 0)

        if nex:
            @pl.when(w < nex)
            def _extra():
                off = (nch * nw + w) * _CHUNK
                pltpu.sync_copy(dst_hbm.at[pl.ds(off, _CHUNK)], dchunk)
                pltpu.sync_copy(ones, cntacc.at[dchunk], add=True)

        plsc.subcore_barrier()

        pltpu.sync_copy(cntacc.at[pl.ds(rbase, rps)],
                        pcnt_hbm.at[ci, pl.ds(rbase, rps)])

    return pl.kernel(
        body, out_type=jax.ShapeDtypeStruct((_NC, npad), jnp.float32),
        mesh=_mesh(), scratch_types=scratch)


def kernel(x, edges, W_down, b_down, W_sc, b_sc, Wl1, bl1, Wr1, ln1_g, ln1_b,
           Wl2, bl2, Wr2, ln2_g, ln2_b):
    n, d = x.shape
    e = edges.shape[1]
    src = edges[0]
    dst = edges[1]
    r = 2000 if n % 2000 == 0 else n

    h0, shortcut = _tc_pre(x, W_down, b_down, W_sc, b_sc, n=n, d=d, r=r)

    cnt = _make_sc_cnt(n, e)(dst)
    c1 = jnp.broadcast_to(cnt[:, :, None], cnt.shape + (_LANES,))
    p1 = _make_sc_agg(n, e, d)(h0, src, dst)
    h1 = _tc_mid(p1, c1, h0, Wl1, bl1, Wr1, ln1_g, ln1_b, n=n, d=d, r=r)

    p2 = _make_sc_agg(n, e, d)(h1, src, dst)
    return _tc_post(p2, c1, h1, Wl2, bl2, Wr2, ln2_g, ln2_b, shortcut,
                    n=n, d=d, r=r)
